# baseline (speedup 1.0000x reference)
import functools

import jax
import jax.numpy as jnp
import numpy as np
from jax.experimental import pallas as pl
from jax.experimental.pallas import tpu as pltpu
from jax.experimental.pallas import tpu_sc as plsc
from jax._src.pallas import mpmd as _pl_mpmd

_B = 4096
_HID = 128
_HEADS = 4
_DH = 32
_K = 8
_R1 = 512
_R2 = 512
_C3 = 256


def _fused_body(f0, f1, f2, w0t, w1t, w2t, pb0, pb1, pb2, g0, b0, g1, b1,
                g2, b2, wiT, bi, woT, bo, e1T, eb1, e2T, eb2, S, ST,
                fused_o, fusedT_o, ew_o):
    def proj(f, wt, pb, g, b):
        x = jnp.dot(f[...], wt[...], preferred_element_type=jnp.float32)
        x = jnp.maximum(x + pb[...], 0.0)
        m = jnp.sum(x, axis=1, keepdims=True) * (1.0 / _HID)
        d = x - m
        v = jnp.sum(d * d, axis=1, keepdims=True) * (1.0 / _HID)
        return d * jax.lax.rsqrt(v + 1e-5) * g[...] + b[...]

    p = [proj(f0, w0t, pb0, g0, b0),
         proj(f1, w1t, pb1, g1, b1),
         proj(f2, w2t, pb2, g2, b2)]
    wi = wiT[...]
    bi_ = bi[...]
    q = [jnp.dot(pm, wi[:, 0:_HID], preferred_element_type=jnp.float32)
         + bi_[:, 0:_HID] for pm in p]
    k = [jnp.dot(pm, wi[:, _HID:2 * _HID], preferred_element_type=jnp.float32)
         + bi_[:, _HID:2 * _HID] for pm in p]
    v = [jnp.dot(pm, wi[:, 2 * _HID:3 * _HID], preferred_element_type=jnp.float32)
         + bi_[:, 2 * _HID:3 * _HID] for pm in p]
    Sm = S[...]
    STm = ST[...]
    scale = 1.0 / np.sqrt(float(_DH))
    osum = None
    for m in range(3):
        s = [jnp.dot(q[m] * k[n], Sm, preferred_element_type=jnp.float32) * scale
             for n in range(3)]
        mx = jnp.maximum(jnp.maximum(s[0], s[1]), s[2])
        e = [jnp.exp(sn - mx) for sn in s]
        den = e[0] + e[1] + e[2]
        o = None
        for n in range(3):
            a = jnp.dot(e[n] / den, STm, preferred_element_type=jnp.float32)
            o = a * v[n] if o is None else o + a * v[n]
        osum = o if osum is None else osum + o
    obar = osum * (1.0 / 3.0)
    fused = jnp.dot(obar, woT[...], preferred_element_type=jnp.float32) + bo[...]
    fused_o[...] = fused
    fusedT_o[...] = fused.T
    h1 = jnp.maximum(
        jnp.dot(fused, e1T[...], preferred_element_type=jnp.float32) + eb1[...],
        0.0)
    z = jnp.dot(h1, e2T[...], preferred_element_type=jnp.float32) + eb2[...]
    ew = 1.0 / (1.0 + jnp.exp(-z))
    ew_o[...] = jnp.maximum(ew, 1e-8)


def _topk_body(fb, ftT, tiT_o, tvT_o):
    l = jnp.dot(fb[...], ftT[...], preferred_element_type=jnp.float32)
    rowmax = jnp.max(l, axis=1, keepdims=True)
    ex = jnp.exp(l - rowmax)
    sumexp = jnp.sum(ex, axis=1, keepdims=True)
    rev = (_B - 1) - jax.lax.broadcasted_iota(jnp.int32, l.shape, 1)
    bits = jax.lax.bitcast_convert_type(ex, jnp.int32)
    keys = jax.lax.bitcast_convert_type(
        ((bits + 0x800) & jnp.int32(~0xFFF)) | rev, jnp.float32)
    j8 = jax.lax.broadcasted_iota(jnp.int32, (l.shape[0], _K), 1)
    topi = jnp.zeros((l.shape[0], _K), jnp.int32)
    topb = jnp.zeros((l.shape[0], _K), jnp.int32)
    for j in range(_K):
        cur = jnp.max(keys, axis=1, keepdims=True)
        if j < _K - 1:
            keys = jnp.where(keys == cur, 0.0, keys)
        curb = jax.lax.bitcast_convert_type(cur, jnp.int32)
        topi = jnp.where(j8 == j, (_B - 1) - (curb & 0xFFF), topi)
        topb = jnp.where(j8 == j, curb & jnp.int32(~0xFFF), topb)
    topv = jax.lax.bitcast_convert_type(topb, jnp.float32) / sumexp
    base = pl.program_id(0) * _R2
    growid = base + jax.lax.broadcasted_iota(jnp.int32, (l.shape[0], 1), 0)
    is_diag = topi == growid
    any_diag = jnp.max(jnp.where(is_diag, 1, 0), axis=1, keepdims=True) > 0
    v_ji = jnp.sum(jnp.where(is_diag, topv, 0.0), axis=1, keepdims=True)
    sumv = jnp.sum(topv, axis=1, keepdims=True)
    diag = jnp.where(any_diag, jnp.where(sumv == 0.0, 1.0, v_ji), 1.0)
    vals = jnp.where(is_diag, diag, topv)
    fill_i = jnp.full((l.shape[0], 16 - (_K + 1)), -1, jnp.int32)
    fill_v = jnp.zeros((l.shape[0], 16 - (_K + 1)), jnp.float32)
    tiT_o[...] = jnp.concatenate([topi, growid, fill_i], axis=1).T
    tvT_o[...] = jnp.concatenate([vals, diag, fill_v], axis=1).T


def _sc_scatter_body(h_hbm, tiT_hbm, tvT_hbm, h_out, ti_v, tv_v, idx_v, sem):
    del h_hbm
    wid = jax.lax.axis_index("s") * 2 + jax.lax.axis_index("c")
    base = wid * 128
    pltpu.sync_copy(tiT_hbm.at[:, pl.ds(base, 128)], ti_v)
    pltpu.sync_copy(tvT_hbm.at[:, pl.ds(base, 128)], tv_v)
    for j in range(_K + 1):
        for c8 in range(8):
            rows = ti_v[j, pl.ds(c8 * 16, 16)]
            cols = base + c8 * 16 + jax.lax.iota(jnp.int32, 16)
            idx_v[j, pl.ds(c8 * 16, 16)] = rows * _B + cols
    copies = [
        pltpu.async_copy(tv_v.at[j], h_out.at[idx_v.at[j]], sem)
        for j in range(_K + 1)
    ]
    for c in copies:
        c.wait()


def kernel(feat0, feat1, feat2, proj_w0, proj_b0, ln_g0, ln_b0, proj_w1,
           proj_b1, ln_g1, ln_b1, proj_w2, proj_b2, ln_g2, ln_b2, in_proj_w,
           in_proj_b, out_proj_w, out_proj_b, ew_w1, ew_b1, ew_w2, ew_b2):
    f32 = jnp.float32
    row = lambda x: x.reshape(1, -1)
    S = (jnp.arange(_HID)[:, None] // _DH ==
         jnp.arange(_HEADS)[None, :]).astype(f32)
    ST = S.T

    grid1 = _B // _R1
    full = lambda shp: pl.BlockSpec(shp, lambda i: (0, 0))
    fused, fusedT, ew = pl.pallas_call(
        _fused_body,
        grid=(grid1,),
        in_specs=[
            pl.BlockSpec((_R1, 512), lambda i: (i, 0)),
            pl.BlockSpec((_R1, 256), lambda i: (i, 0)),
            pl.BlockSpec((_R1, 128), lambda i: (i, 0)),
            full((512, _HID)), full((256, _HID)), full((128, _HID)),
            full((1, _HID)), full((1, _HID)), full((1, _HID)),
            full((1, _HID)), full((1, _HID)), full((1, _HID)),
            full((1, _HID)), full((1, _HID)), full((1, _HID)),
            full((_HID, 3 * _HID)), full((1, 3 * _HID)),
            full((_HID, _HID)), full((1, _HID)),
            full((_HID, _HID // 2)), full((1, _HID // 2)),
            full((_HID // 2, 1)), full((1, 1)),
            full((_HID, _HEADS)), full((_HEADS, _HID)),
        ],
        out_specs=[
            pl.BlockSpec((_R1, _HID), lambda i: (i, 0)),
            pl.BlockSpec((_HID, _R1), lambda i: (0, i)),
            pl.BlockSpec((_R1, 1), lambda i: (i, 0)),
        ],
        out_shape=[
            jax.ShapeDtypeStruct((_B, _HID), f32),
            jax.ShapeDtypeStruct((_HID, _B), f32),
            jax.ShapeDtypeStruct((_B, 1), f32),
        ],
    )(feat0, feat1, feat2,
      proj_w0.T, proj_w1.T, proj_w2.T,
      row(proj_b0), row(proj_b1), row(proj_b2),
      row(ln_g0), row(ln_b0), row(ln_g1), row(ln_b1), row(ln_g2), row(ln_b2),
      in_proj_w.T, row(in_proj_b), out_proj_w.T, row(out_proj_b),
      ew_w1.T, row(ew_b1), ew_w2.T, row(ew_b2), S, ST)

    grid2 = _B // _R2
    tiT, tvT = pl.pallas_call(
        _topk_body,
        grid=(grid2,),
        in_specs=[
            pl.BlockSpec((_R2, _HID), lambda i: (i, 0)),
            pl.BlockSpec((_HID, _B), lambda i: (0, 0)),
        ],
        out_specs=[
            pl.BlockSpec((16, _R2), lambda i: (0, i)),
            pl.BlockSpec((16, _R2), lambda i: (0, i)),
        ],
        out_shape=[
            jax.ShapeDtypeStruct((16, _B), jnp.int32),
            jax.ShapeDtypeStruct((16, _B), f32),
        ],
    )(fused, fusedT)

    h0 = jnp.zeros((_B * _B,), f32)
    mesh = plsc.VectorSubcoreMesh(core_axis_name="c", subcore_axis_name="s")
    scatter = _pl_mpmd._mpmd_map(
        [(mesh, _sc_scatter_body)],
        jax.ShapeDtypeStruct((_B * _B,), f32),
        input_output_aliases={0: 0},
        scratch_types=[
            pltpu.VMEM((16, 128), jnp.int32),
            pltpu.VMEM((16, 128), f32),
            pltpu.VMEM((_K + 1, 128), jnp.int32),
            pltpu.SemaphoreType.DMA,
        ],
    )
    Hm = scatter(h0, tiT, tvT).reshape(_B, _B)

    return (Hm, ew.reshape(_B))


# speedup vs baseline: 1.6196x; 1.6196x over previous
import jax
import jax.numpy as jnp
import numpy as np
from jax.experimental import pallas as pl

_B = 4096
_HID = 128
_HEADS = 4
_DH = 32
_K = 8
_R1 = 512
_R2 = 512
_C3 = 256


def _fused_body(f0, f1, f2, w0t, w1t, w2t, pb0, pb1, pb2, g0, b0, g1, b1,
                g2, b2, wiT, bi, woT, bo, e1T, eb1, e2T, eb2, S, ST,
                fused_o, fusedT_o, ew_o):
    def proj(f, wt, pb, g, b):
        x = jnp.dot(f[...], wt[...], preferred_element_type=jnp.float32)
        x = jnp.maximum(x + pb[...], 0.0)
        m = jnp.sum(x, axis=1, keepdims=True) * (1.0 / _HID)
        d = x - m
        v = jnp.sum(d * d, axis=1, keepdims=True) * (1.0 / _HID)
        return d * jax.lax.rsqrt(v + 1e-5) * g[...] + b[...]

    p = [proj(f0, w0t, pb0, g0, b0),
         proj(f1, w1t, pb1, g1, b1),
         proj(f2, w2t, pb2, g2, b2)]
    wi = wiT[...]
    bi_ = bi[...]
    q = [jnp.dot(pm, wi[:, 0:_HID], preferred_element_type=jnp.float32)
         + bi_[:, 0:_HID] for pm in p]
    k = [jnp.dot(pm, wi[:, _HID:2 * _HID], preferred_element_type=jnp.float32)
         + bi_[:, _HID:2 * _HID] for pm in p]
    v = [jnp.dot(pm, wi[:, 2 * _HID:3 * _HID], preferred_element_type=jnp.float32)
         + bi_[:, 2 * _HID:3 * _HID] for pm in p]
    Sm = S[...]
    STm = ST[...]
    scale = 1.0 / np.sqrt(float(_DH))
    osum = None
    for m in range(3):
        s = [jnp.dot(q[m] * k[n], Sm, preferred_element_type=jnp.float32) * scale
             for n in range(3)]
        mx = jnp.maximum(jnp.maximum(s[0], s[1]), s[2])
        e = [jnp.exp(sn - mx) for sn in s]
        den = e[0] + e[1] + e[2]
        o = None
        for n in range(3):
            a = jnp.dot(e[n] / den, STm, preferred_element_type=jnp.float32)
            o = a * v[n] if o is None else o + a * v[n]
        osum = o if osum is None else osum + o
    obar = osum * (1.0 / 3.0)
    fused = jnp.dot(obar, woT[...], preferred_element_type=jnp.float32) + bo[...]
    fused_o[...] = fused
    fusedT_o[...] = fused.T
    h1 = jnp.maximum(
        jnp.dot(fused, e1T[...], preferred_element_type=jnp.float32) + eb1[...],
        0.0)
    z = jnp.dot(h1, e2T[...], preferred_element_type=jnp.float32) + eb2[...]
    ew = 1.0 / (1.0 + jnp.exp(-z))
    ew_o[...] = jnp.maximum(ew, 1e-8)


def _topk_body(fb, ftT, tiT_o, tvT_o):
    l = jnp.dot(fb[...], ftT[...], preferred_element_type=jnp.float32)
    rowmax = jnp.max(l, axis=1, keepdims=True)
    ex = jnp.exp(l - rowmax)
    sumexp = jnp.sum(ex, axis=1, keepdims=True)
    rev = (_B - 1) - jax.lax.broadcasted_iota(jnp.int32, l.shape, 1)
    bits = jax.lax.bitcast_convert_type(ex, jnp.int32)
    keys = jax.lax.bitcast_convert_type(
        ((bits + 0x800) & jnp.int32(~0xFFF)) | rev, jnp.float32)
    j8 = jax.lax.broadcasted_iota(jnp.int32, (l.shape[0], _K), 1)
    topi = jnp.zeros((l.shape[0], _K), jnp.int32)
    topb = jnp.zeros((l.shape[0], _K), jnp.int32)
    for j in range(_K):
        cur = jnp.max(keys, axis=1, keepdims=True)
        if j < _K - 1:
            keys = jnp.where(keys == cur, 0.0, keys)
        curb = jax.lax.bitcast_convert_type(cur, jnp.int32)
        topi = jnp.where(j8 == j, (_B - 1) - (curb & 0xFFF), topi)
        topb = jnp.where(j8 == j, curb & jnp.int32(~0xFFF), topb)
    topv = jax.lax.bitcast_convert_type(topb, jnp.float32) / sumexp
    base = pl.program_id(0) * _R2
    growid = base + jax.lax.broadcasted_iota(jnp.int32, (l.shape[0], 1), 0)
    is_diag = topi == growid
    any_diag = jnp.max(jnp.where(is_diag, 1, 0), axis=1, keepdims=True) > 0
    v_ji = jnp.sum(jnp.where(is_diag, topv, 0.0), axis=1, keepdims=True)
    sumv = jnp.sum(topv, axis=1, keepdims=True)
    diag = jnp.where(any_diag, jnp.where(sumv == 0.0, 1.0, v_ji), 1.0)
    vals = jnp.where(is_diag, diag, topv)
    fill_i = jnp.full((l.shape[0], 16 - (_K + 1)), -1, jnp.int32)
    fill_v = jnp.zeros((l.shape[0], 16 - (_K + 1)), jnp.float32)
    tiT_o[...] = jnp.concatenate([topi, growid, fill_i], axis=1).T
    tvT_o[...] = jnp.concatenate([vals, diag, fill_v], axis=1).T


def _scatter_body(tiT, tvT, h_o):
    riota = jax.lax.broadcasted_iota(jnp.int32, (_B, _C3), 0)
    h = jnp.zeros((_B, _C3), jnp.float32)
    for j in range(_K + 1):
        h = jnp.where(riota == tiT[j:j + 1, :], tvT[j:j + 1, :], h)
    h_o[...] = h


def kernel(feat0, feat1, feat2, proj_w0, proj_b0, ln_g0, ln_b0, proj_w1,
           proj_b1, ln_g1, ln_b1, proj_w2, proj_b2, ln_g2, ln_b2, in_proj_w,
           in_proj_b, out_proj_w, out_proj_b, ew_w1, ew_b1, ew_w2, ew_b2):
    f32 = jnp.float32
    row = lambda x: x.reshape(1, -1)
    S = (jnp.arange(_HID)[:, None] // _DH ==
         jnp.arange(_HEADS)[None, :]).astype(f32)
    ST = S.T

    grid1 = _B // _R1
    full = lambda shp: pl.BlockSpec(shp, lambda i: (0, 0))
    fused, fusedT, ew = pl.pallas_call(
        _fused_body,
        grid=(grid1,),
        in_specs=[
            pl.BlockSpec((_R1, 512), lambda i: (i, 0)),
            pl.BlockSpec((_R1, 256), lambda i: (i, 0)),
            pl.BlockSpec((_R1, 128), lambda i: (i, 0)),
            full((512, _HID)), full((256, _HID)), full((128, _HID)),
            full((1, _HID)), full((1, _HID)), full((1, _HID)),
            full((1, _HID)), full((1, _HID)), full((1, _HID)),
            full((1, _HID)), full((1, _HID)), full((1, _HID)),
            full((_HID, 3 * _HID)), full((1, 3 * _HID)),
            full((_HID, _HID)), full((1, _HID)),
            full((_HID, _HID // 2)), full((1, _HID // 2)),
            full((_HID // 2, 1)), full((1, 1)),
            full((_HID, _HEADS)), full((_HEADS, _HID)),
        ],
        out_specs=[
            pl.BlockSpec((_R1, _HID), lambda i: (i, 0)),
            pl.BlockSpec((_HID, _R1), lambda i: (0, i)),
            pl.BlockSpec((_R1, 1), lambda i: (i, 0)),
        ],
        out_shape=[
            jax.ShapeDtypeStruct((_B, _HID), f32),
            jax.ShapeDtypeStruct((_HID, _B), f32),
            jax.ShapeDtypeStruct((_B, 1), f32),
        ],
    )(feat0, feat1, feat2,
      proj_w0.T, proj_w1.T, proj_w2.T,
      row(proj_b0), row(proj_b1), row(proj_b2),
      row(ln_g0), row(ln_b0), row(ln_g1), row(ln_b1), row(ln_g2), row(ln_b2),
      in_proj_w.T, row(in_proj_b), out_proj_w.T, row(out_proj_b),
      ew_w1.T, row(ew_b1), ew_w2.T, row(ew_b2), S, ST)

    grid2 = _B // _R2
    tiT, tvT = pl.pallas_call(
        _topk_body,
        grid=(grid2,),
        in_specs=[
            pl.BlockSpec((_R2, _HID), lambda i: (i, 0)),
            pl.BlockSpec((_HID, _B), lambda i: (0, 0)),
        ],
        out_specs=[
            pl.BlockSpec((16, _R2), lambda i: (0, i)),
            pl.BlockSpec((16, _R2), lambda i: (0, i)),
        ],
        out_shape=[
            jax.ShapeDtypeStruct((16, _B), jnp.int32),
            jax.ShapeDtypeStruct((16, _B), f32),
        ],
    )(fused, fusedT)

    grid3 = _B // _C3
    Hm = pl.pallas_call(
        _scatter_body,
        grid=(grid3,),
        in_specs=[
            pl.BlockSpec((16, _C3), lambda j: (0, j)),
            pl.BlockSpec((16, _C3), lambda j: (0, j)),
        ],
        out_specs=pl.BlockSpec((_B, _C3), lambda j: (0, j)),
        out_shape=jax.ShapeDtypeStruct((_B, _B), f32),
    )(tiT, tvT)

    return (Hm, ew.reshape(_B))


# speedup vs baseline: 1.6511x; 1.0194x over previous
import jax
import jax.numpy as jnp
import numpy as np
from jax.experimental import pallas as pl

_B = 4096
_HID = 128
_HEADS = 4
_DH = 32
_K = 8
_R1 = 1024
_R2 = 512
_C3 = 256


def _fused_body(f0, f1, f2, w0t, w1t, w2t, pb0, pb1, pb2, g0, b0, g1, b1,
                g2, b2, wiT, bi, woT, bo, e1T, eb1, e2T, eb2, S, ST,
                fused_o, fusedT_o, ew_o):
    def proj(f, wt, pb, g, b):
        x = jnp.dot(f[...], wt[...], preferred_element_type=jnp.float32)
        x = jnp.maximum(x + pb[...], 0.0)
        m = jnp.sum(x, axis=1, keepdims=True) * (1.0 / _HID)
        d = x - m
        v = jnp.sum(d * d, axis=1, keepdims=True) * (1.0 / _HID)
        return d * jax.lax.rsqrt(v + 1e-5) * g[...] + b[...]

    p = [proj(f0, w0t, pb0, g0, b0),
         proj(f1, w1t, pb1, g1, b1),
         proj(f2, w2t, pb2, g2, b2)]
    wi = wiT[...]
    bi_ = bi[...]
    q = [jnp.dot(pm, wi[:, 0:_HID], preferred_element_type=jnp.float32)
         + bi_[:, 0:_HID] for pm in p]
    k = [jnp.dot(pm, wi[:, _HID:2 * _HID], preferred_element_type=jnp.float32)
         + bi_[:, _HID:2 * _HID] for pm in p]
    v = [jnp.dot(pm, wi[:, 2 * _HID:3 * _HID], preferred_element_type=jnp.float32)
         + bi_[:, 2 * _HID:3 * _HID] for pm in p]
    Sm = S[...]
    STm = ST[...]
    scale = 1.0 / np.sqrt(float(_DH))
    osum = None
    for m in range(3):
        s = [jnp.dot(q[m] * k[n], Sm, preferred_element_type=jnp.float32) * scale
             for n in range(3)]
        mx = jnp.maximum(jnp.maximum(s[0], s[1]), s[2])
        e = [jnp.exp(sn - mx) for sn in s]
        den = e[0] + e[1] + e[2]
        o = None
        for n in range(3):
            a = jnp.dot(e[n] / den, STm, preferred_element_type=jnp.float32)
            o = a * v[n] if o is None else o + a * v[n]
        osum = o if osum is None else osum + o
    obar = osum * (1.0 / 3.0)
    fused = jnp.dot(obar, woT[...], preferred_element_type=jnp.float32) + bo[...]
    fused_o[...] = fused
    fusedT_o[...] = fused.T
    h1 = jnp.maximum(
        jnp.dot(fused, e1T[...], preferred_element_type=jnp.float32) + eb1[...],
        0.0)
    z = jnp.dot(h1, e2T[...], preferred_element_type=jnp.float32) + eb2[...]
    ew = 1.0 / (1.0 + jnp.exp(-z))
    ew_o[...] = jnp.maximum(ew, 1e-8)


def _topk_body(fb, ftT, tiT_o, tvT_o):
    l = jnp.dot(fb[...], ftT[...], preferred_element_type=jnp.float32)
    rowmax = jnp.max(l, axis=1, keepdims=True)
    ex = jnp.exp(l - rowmax)
    sumexp = jnp.sum(ex, axis=1, keepdims=True)
    rev = (_B - 1) - jax.lax.broadcasted_iota(jnp.int32, l.shape, 1)
    bits = jax.lax.bitcast_convert_type(ex, jnp.int32)
    keys = jax.lax.bitcast_convert_type(
        ((bits + 0x800) & jnp.int32(~0xFFF)) | rev, jnp.float32)
    j8 = jax.lax.broadcasted_iota(jnp.int32, (l.shape[0], _K), 1)
    topi = jnp.zeros((l.shape[0], _K), jnp.int32)
    topb = jnp.zeros((l.shape[0], _K), jnp.int32)
    for j in range(_K):
        cur = jnp.max(keys, axis=1, keepdims=True)
        if j < _K - 1:
            keys = jnp.where(keys == cur, 0.0, keys)
        curb = jax.lax.bitcast_convert_type(cur, jnp.int32)
        topi = jnp.where(j8 == j, (_B - 1) - (curb & 0xFFF), topi)
        topb = jnp.where(j8 == j, curb & jnp.int32(~0xFFF), topb)
    topv = jax.lax.bitcast_convert_type(topb, jnp.float32) / sumexp
    base = pl.program_id(0) * _R2
    growid = base + jax.lax.broadcasted_iota(jnp.int32, (l.shape[0], 1), 0)
    is_diag = topi == growid
    any_diag = jnp.max(jnp.where(is_diag, 1, 0), axis=1, keepdims=True) > 0
    v_ji = jnp.sum(jnp.where(is_diag, topv, 0.0), axis=1, keepdims=True)
    sumv = jnp.sum(topv, axis=1, keepdims=True)
    diag = jnp.where(any_diag, jnp.where(sumv == 0.0, 1.0, v_ji), 1.0)
    vals = jnp.where(is_diag, diag, topv)
    fill_i = jnp.full((l.shape[0], 16 - (_K + 1)), -1, jnp.int32)
    fill_v = jnp.zeros((l.shape[0], 16 - (_K + 1)), jnp.float32)
    tiT_o[...] = jnp.concatenate([topi, growid, fill_i], axis=1).T
    tvT_o[...] = jnp.concatenate([vals, diag, fill_v], axis=1).T


def _scatter_body(tiT, tvT, h_o):
    riota = jax.lax.broadcasted_iota(jnp.int32, (_B, _C3), 0)
    h = jnp.zeros((_B, _C3), jnp.float32)
    for j in range(_K + 1):
        h = jnp.where(riota == tiT[j:j + 1, :], tvT[j:j + 1, :], h)
    h_o[...] = h


def kernel(feat0, feat1, feat2, proj_w0, proj_b0, ln_g0, ln_b0, proj_w1,
           proj_b1, ln_g1, ln_b1, proj_w2, proj_b2, ln_g2, ln_b2, in_proj_w,
           in_proj_b, out_proj_w, out_proj_b, ew_w1, ew_b1, ew_w2, ew_b2):
    f32 = jnp.float32
    row = lambda x: x.reshape(1, -1)
    S = (jnp.arange(_HID)[:, None] // _DH ==
         jnp.arange(_HEADS)[None, :]).astype(f32)
    ST = S.T

    grid1 = _B // _R1
    full = lambda shp: pl.BlockSpec(shp, lambda i: (0, 0))
    fused, fusedT, ew = pl.pallas_call(
        _fused_body,
        grid=(grid1,),
        in_specs=[
            pl.BlockSpec((_R1, 512), lambda i: (i, 0)),
            pl.BlockSpec((_R1, 256), lambda i: (i, 0)),
            pl.BlockSpec((_R1, 128), lambda i: (i, 0)),
            full((512, _HID)), full((256, _HID)), full((128, _HID)),
            full((1, _HID)), full((1, _HID)), full((1, _HID)),
            full((1, _HID)), full((1, _HID)), full((1, _HID)),
            full((1, _HID)), full((1, _HID)), full((1, _HID)),
            full((_HID, 3 * _HID)), full((1, 3 * _HID)),
            full((_HID, _HID)), full((1, _HID)),
            full((_HID, _HID // 2)), full((1, _HID // 2)),
            full((_HID // 2, 1)), full((1, 1)),
            full((_HID, _HEADS)), full((_HEADS, _HID)),
        ],
        out_specs=[
            pl.BlockSpec((_R1, _HID), lambda i: (i, 0)),
            pl.BlockSpec((_HID, _R1), lambda i: (0, i)),
            pl.BlockSpec((_R1, 1), lambda i: (i, 0)),
        ],
        out_shape=[
            jax.ShapeDtypeStruct((_B, _HID), f32),
            jax.ShapeDtypeStruct((_HID, _B), f32),
            jax.ShapeDtypeStruct((_B, 1), f32),
        ],
    )(feat0, feat1, feat2,
      proj_w0.T, proj_w1.T, proj_w2.T,
      row(proj_b0), row(proj_b1), row(proj_b2),
      row(ln_g0), row(ln_b0), row(ln_g1), row(ln_b1), row(ln_g2), row(ln_b2),
      in_proj_w.T, row(in_proj_b), out_proj_w.T, row(out_proj_b),
      ew_w1.T, row(ew_b1), ew_w2.T, row(ew_b2), S, ST)

    grid2 = _B // _R2
    tiT, tvT = pl.pallas_call(
        _topk_body,
        grid=(grid2,),
        in_specs=[
            pl.BlockSpec((_R2, _HID), lambda i: (i, 0)),
            pl.BlockSpec((_HID, _B), lambda i: (0, 0)),
        ],
        out_specs=[
            pl.BlockSpec((16, _R2), lambda i: (0, i)),
            pl.BlockSpec((16, _R2), lambda i: (0, i)),
        ],
        out_shape=[
            jax.ShapeDtypeStruct((16, _B), jnp.int32),
            jax.ShapeDtypeStruct((16, _B), f32),
        ],
    )(fused, fusedT)

    grid3 = _B // _C3
    Hm = pl.pallas_call(
        _scatter_body,
        grid=(grid3,),
        in_specs=[
            pl.BlockSpec((16, _C3), lambda j: (0, j)),
            pl.BlockSpec((16, _C3), lambda j: (0, j)),
        ],
        out_specs=pl.BlockSpec((_B, _C3), lambda j: (0, j)),
        out_shape=jax.ShapeDtypeStruct((_B, _B), f32),
    )(tiT, tvT)

    return (Hm, ew.reshape(_B))
